# baseline (device time: 49060 ns/iter reference)
import jax
import jax.numpy as jnp
from jax import lax
from jax.experimental import pallas as pl
from jax.experimental.pallas import tpu as pltpu

N_DEV = 4


def kernel(dy, W):
    m, k = dy.shape
    d = W.shape[0]

    def body(dy_ref, w_ref, out_ref, comm_ref, send_sems, recv_sems):
        my_pos = lax.axis_index("i")
        left = (my_pos - 1) % N_DEV
        right = (my_pos + 1) % N_DEV

        barrier_sem = pltpu.get_barrier_semaphore()
        for nbr in [left, right]:
            pl.semaphore_signal(
                barrier_sem, inc=1,
                device_id=(nbr,), device_id_type=pl.DeviceIdType.MESH,
            )
        pl.semaphore_wait(barrier_sem, 2)

        partial = lax.dot_general(
            dy_ref[...], w_ref[...],
            dimension_numbers=(((1,), (1,)), ((), ())),
            preferred_element_type=jnp.float32,
        )
        out_ref[...] = partial
        comm_ref[0] = partial

        for h in range(N_DEV - 1):
            rdma = pltpu.make_async_remote_copy(
                src_ref=comm_ref.at[h],
                dst_ref=comm_ref.at[h + 1],
                send_sem=send_sems.at[h],
                recv_sem=recv_sems.at[h + 1],
                device_id=(right,),
                device_id_type=pl.DeviceIdType.MESH,
            )
            rdma.start()
            rdma.wait()
            out_ref[...] += comm_ref[h + 1]

    return pl.pallas_call(
        body,
        out_shape=jax.ShapeDtypeStruct((m, d), jnp.float32),
        in_specs=[
            pl.BlockSpec(memory_space=pltpu.VMEM),
            pl.BlockSpec(memory_space=pltpu.VMEM),
        ],
        out_specs=pl.BlockSpec(memory_space=pltpu.VMEM),
        scratch_shapes=[
            pltpu.VMEM((N_DEV, m, d), jnp.float32),
            pltpu.SemaphoreType.DMA((N_DEV,)),
            pltpu.SemaphoreType.DMA((N_DEV,)),
        ],
        compiler_params=pltpu.CompilerParams(collective_id=0),
    )(dy, W)


# device time: 24906 ns/iter; 1.9698x vs baseline; 1.9698x over previous
import jax
import jax.numpy as jnp
from jax import lax
from jax.experimental import pallas as pl
from jax.experimental.pallas import tpu as pltpu

N_DEV = 4


def kernel(dy, W):
    m, k = dy.shape
    d = W.shape[0]

    def body(dy_ref, w_ref, out_ref, comm1, comm2, send_sems, recv_sems):
        p = lax.axis_index("i")
        a = jnp.bitwise_xor(p, 1)
        b = 3 - p

        barrier_sem = pltpu.get_barrier_semaphore()
        for nbr in [a, b]:
            pl.semaphore_signal(
                barrier_sem, inc=1,
                device_id=(nbr,), device_id_type=pl.DeviceIdType.MESH,
            )
        pl.semaphore_wait(barrier_sem, 2)

        out_ref[...] = lax.dot_general(
            dy_ref[...], w_ref[...],
            dimension_numbers=(((1,), (1,)), ((), ())),
            preferred_element_type=jnp.float32,
        )

        p_lt2 = p < 2
        p_even = (p % 2) == 0
        p_03 = (p == 0) | (p == 3)

        k1 = jnp.where(p_03, 0, 128)
        o2 = jnp.where(p_lt2, 0, 64)
        kB = jnp.where(p_lt2, 0, 128)
        oB = jnp.where(p_even, 0, 64)

        def exch(idx, partner, src_off, nrows, dst):
            r = pltpu.make_async_remote_copy(
                src_ref=out_ref.at[pl.ds(src_off, nrows), :],
                dst_ref=dst,
                send_sem=send_sems.at[idx],
                recv_sem=recv_sems.at[idx],
                device_id=(partner,),
                device_id_type=pl.DeviceIdType.MESH,
            )
            r.start()
            return r

        r0 = exch(0, a, 128 - k1, 128, comm1.at[0])
        r1 = exch(1, b, 384 - kB, 128, comm1.at[1])
        r0.wait_recv()
        out_ref[pl.ds(k1, 128), :] += comm1[0]
        r1.wait_recv()
        out_ref[pl.ds(256 + kB, 128), :] += comm1[1]
        r0.wait_send()
        r1.wait_send()

        r2 = exch(2, b, k1 + 64 - o2, 64, comm2.at[0])
        r3 = exch(3, a, 256 + kB + 64 - oB, 64, comm2.at[1])
        r2.wait_recv()
        out_ref[pl.ds(k1 + o2, 64), :] += comm2[0]
        r3.wait_recv()
        out_ref[pl.ds(256 + kB + oB, 64), :] += comm2[1]
        r2.wait_send()
        r3.wait_send()

        r4 = exch(4, b, k1 + o2, 64,
                  out_ref.at[pl.ds(k1 + o2, 64), :])
        r5 = exch(5, a, 256 + kB + oB, 64,
                  out_ref.at[pl.ds(256 + kB + oB, 64), :])
        r4.wait_recv()
        r5.wait_recv()
        r4.wait_send()
        r5.wait_send()

        r6 = exch(6, a, k1, 128, out_ref.at[pl.ds(k1, 128), :])
        r7 = exch(7, b, 256 + kB, 128, out_ref.at[pl.ds(256 + kB, 128), :])
        r6.wait_recv()
        r7.wait_recv()
        r6.wait_send()
        r7.wait_send()

    return pl.pallas_call(
        body,
        out_shape=jax.ShapeDtypeStruct((m, d), jnp.float32),
        in_specs=[
            pl.BlockSpec(memory_space=pltpu.VMEM),
            pl.BlockSpec(memory_space=pltpu.VMEM),
        ],
        out_specs=pl.BlockSpec(memory_space=pltpu.VMEM),
        scratch_shapes=[
            pltpu.VMEM((2, 128, d), jnp.float32),
            pltpu.VMEM((2, 64, d), jnp.float32),
            pltpu.SemaphoreType.DMA((8,)),
            pltpu.SemaphoreType.DMA((8,)),
        ],
        compiler_params=pltpu.CompilerParams(collective_id=0),
    )(dy, W)


# device time: 23816 ns/iter; 2.0600x vs baseline; 1.0458x over previous
import jax
import jax.numpy as jnp
from jax import lax
from jax.experimental import pallas as pl
from jax.experimental.pallas import tpu as pltpu

N_DEV = 4


def kernel(dy, W):
    m, k = dy.shape
    d = W.shape[0]

    def body(dy_ref, w_ref, out_ref, comm1, comm2, send_sems, recv_sems):
        p = lax.axis_index("i")
        a = jnp.bitwise_xor(p, 1)
        b = 3 - p

        barrier_sem = pltpu.get_barrier_semaphore()
        for nbr in [a, b]:
            pl.semaphore_signal(
                barrier_sem, inc=1,
                device_id=(nbr,), device_id_type=pl.DeviceIdType.MESH,
            )
        pl.semaphore_wait(barrier_sem, 2)

        p_lt2 = p < 2
        p_even = (p % 2) == 0
        p_03 = (p == 0) | (p == 3)

        k1 = jnp.where(p_03, 0, 128)
        o2 = jnp.where(p_lt2, 0, 64)
        kB = jnp.where(p_lt2, 0, 128)
        oB = jnp.where(p_even, 0, 64)

        def gemm_rows(off):
            out_ref[pl.ds(off, 128), :] = lax.dot_general(
                dy_ref[pl.ds(off, 128), :], w_ref[...],
                dimension_numbers=(((1,), (1,)), ((), ())),
                preferred_element_type=jnp.float32,
            )

        def exch(idx, partner, src_off, nrows, dst):
            r = pltpu.make_async_remote_copy(
                src_ref=out_ref.at[pl.ds(src_off, nrows), :],
                dst_ref=dst,
                send_sem=send_sems.at[idx],
                recv_sem=recv_sems.at[idx],
                device_id=(partner,),
                device_id_type=pl.DeviceIdType.MESH,
            )
            r.start()
            return r

        gemm_rows(128 - k1)
        gemm_rows(384 - kB)

        r0 = exch(0, a, 128 - k1, 128, comm1.at[0])
        r1 = exch(1, b, 384 - kB, 128, comm1.at[1])

        gemm_rows(k1)
        gemm_rows(256 + kB)

        r0.wait_recv()
        r1.wait_recv()
        out_ref[pl.ds(k1 + 64 - o2, 64), :] += comm1[0, pl.ds(64 - o2, 64), :]
        out_ref[pl.ds(256 + kB + 64 - oB, 64), :] += comm1[1, pl.ds(64 - oB, 64), :]

        r2 = exch(2, b, k1 + 64 - o2, 64, comm2.at[0])
        r3 = exch(3, a, 256 + kB + 64 - oB, 64, comm2.at[1])

        out_ref[pl.ds(k1 + o2, 64), :] += comm1[0, pl.ds(o2, 64), :]
        out_ref[pl.ds(256 + kB + oB, 64), :] += comm1[1, pl.ds(oB, 64), :]

        r2.wait_recv()
        out_ref[pl.ds(k1 + o2, 64), :] += comm2[0]
        r3.wait_recv()
        out_ref[pl.ds(256 + kB + oB, 64), :] += comm2[1]

        r4 = exch(4, b, k1 + o2, 64, out_ref.at[pl.ds(k1 + o2, 64), :])
        r5 = exch(5, a, 256 + kB + oB, 64,
                  out_ref.at[pl.ds(256 + kB + oB, 64), :])
        r6 = exch(6, a, k1 + o2, 64, out_ref.at[pl.ds(k1 + o2, 64), :])
        r7 = exch(7, b, 256 + kB + oB, 64,
                  out_ref.at[pl.ds(256 + kB + oB, 64), :])

        r4.wait_recv()
        r8 = exch(8, a, k1 + 64 - o2, 64,
                  out_ref.at[pl.ds(k1 + 64 - o2, 64), :])
        r5.wait_recv()
        r9 = exch(9, b, 256 + kB + 64 - oB, 64,
                  out_ref.at[pl.ds(256 + kB + 64 - oB, 64), :])

        r6.wait_recv()
        r7.wait_recv()
        r8.wait_recv()
        r9.wait_recv()
        for r in (r0, r1, r2, r3, r4, r5, r6, r7, r8, r9):
            r.wait_send()

    return pl.pallas_call(
        body,
        out_shape=jax.ShapeDtypeStruct((m, d), jnp.float32),
        in_specs=[
            pl.BlockSpec(memory_space=pltpu.VMEM),
            pl.BlockSpec(memory_space=pltpu.VMEM),
        ],
        out_specs=pl.BlockSpec(memory_space=pltpu.VMEM),
        scratch_shapes=[
            pltpu.VMEM((2, 128, d), jnp.float32),
            pltpu.VMEM((2, 64, d), jnp.float32),
            pltpu.SemaphoreType.DMA((10,)),
            pltpu.SemaphoreType.DMA((10,)),
        ],
        compiler_params=pltpu.CompilerParams(collective_id=0),
    )(dy, W)


# device time: 23365 ns/iter; 2.0997x vs baseline; 1.0193x over previous
import jax
import jax.numpy as jnp
from jax import lax
from jax.experimental import pallas as pl
from jax.experimental.pallas import tpu as pltpu

N_DEV = 4


def kernel(dy, W):
    m, k = dy.shape
    d = W.shape[0]

    def body(dy_ref, w_ref, out_ref, comm1, comm2, send_sems, recv_sems):
        p = lax.axis_index("i")
        a = jnp.bitwise_xor(p, 1)
        b = 3 - p

        barrier_sem = pltpu.get_barrier_semaphore()
        for nbr in [a, b]:
            pl.semaphore_signal(
                barrier_sem, inc=1,
                device_id=(nbr,), device_id_type=pl.DeviceIdType.MESH,
            )

        p_lt2 = p < 2
        p_even = (p % 2) == 0
        p_03 = (p == 0) | (p == 3)

        k1 = jnp.where(p_03, 0, 128)
        o2 = jnp.where(p_lt2, 0, 64)
        kB = jnp.where(p_lt2, 0, 128)
        oB = jnp.where(p_even, 0, 64)

        def gemm_rows(off):
            out_ref[pl.ds(off, 128), :] = lax.dot_general(
                dy_ref[pl.ds(off, 128), :], w_ref[...],
                dimension_numbers=(((1,), (1,)), ((), ())),
                preferred_element_type=jnp.float32,
            )

        def exch(idx, partner, src_off, nrows, dst):
            r = pltpu.make_async_remote_copy(
                src_ref=out_ref.at[pl.ds(src_off, nrows), :],
                dst_ref=dst,
                send_sem=send_sems.at[idx],
                recv_sem=recv_sems.at[idx],
                device_id=(partner,),
                device_id_type=pl.DeviceIdType.MESH,
            )
            r.start()
            return r

        gemm_rows(128 - k1)
        gemm_rows(384 - kB)

        pl.semaphore_wait(barrier_sem, 2)

        r0 = exch(0, a, 128 - k1, 128, comm1.at[0])
        r1 = exch(1, b, 384 - kB, 128, comm1.at[1])

        gemm_rows(k1)
        gemm_rows(256 + kB)

        r0.wait_recv()
        r1.wait_recv()
        out_ref[pl.ds(k1 + 64 - o2, 64), :] += comm1[0, pl.ds(64 - o2, 64), :]
        out_ref[pl.ds(256 + kB + 64 - oB, 64), :] += comm1[1, pl.ds(64 - oB, 64), :]

        r2 = exch(2, b, k1 + 64 - o2, 64, comm2.at[0])
        r3 = exch(3, a, 256 + kB + 64 - oB, 64, comm2.at[1])

        out_ref[pl.ds(k1 + o2, 64), :] += comm1[0, pl.ds(o2, 64), :]
        out_ref[pl.ds(256 + kB + oB, 64), :] += comm1[1, pl.ds(oB, 64), :]

        r2.wait_recv()
        out_ref[pl.ds(k1 + o2, 64), :] += comm2[0]
        r3.wait_recv()
        out_ref[pl.ds(256 + kB + oB, 64), :] += comm2[1]

        r4 = exch(4, b, k1 + o2, 64, out_ref.at[pl.ds(k1 + o2, 64), :])
        r5 = exch(5, a, 256 + kB + oB, 64,
                  out_ref.at[pl.ds(256 + kB + oB, 64), :])
        r6 = exch(6, a, k1 + o2, 64, out_ref.at[pl.ds(k1 + o2, 64), :])
        r7 = exch(7, b, 256 + kB + oB, 64,
                  out_ref.at[pl.ds(256 + kB + oB, 64), :])

        r4.wait_recv()
        r8 = exch(8, a, k1 + 64 - o2, 64,
                  out_ref.at[pl.ds(k1 + 64 - o2, 64), :])
        r5.wait_recv()
        r9 = exch(9, b, 256 + kB + 64 - oB, 64,
                  out_ref.at[pl.ds(256 + kB + 64 - oB, 64), :])

        r6.wait_recv()
        r7.wait_recv()
        r8.wait_recv()
        r9.wait_recv()
        for r in (r0, r1, r2, r3, r4, r5, r6, r7, r8, r9):
            r.wait_send()

    return pl.pallas_call(
        body,
        out_shape=jax.ShapeDtypeStruct((m, d), jnp.float32),
        in_specs=[
            pl.BlockSpec(memory_space=pltpu.VMEM),
            pl.BlockSpec(memory_space=pltpu.VMEM),
        ],
        out_specs=pl.BlockSpec(memory_space=pltpu.VMEM),
        scratch_shapes=[
            pltpu.VMEM((2, 128, d), jnp.float32),
            pltpu.VMEM((2, 64, d), jnp.float32),
            pltpu.SemaphoreType.DMA((10,)),
            pltpu.SemaphoreType.DMA((10,)),
        ],
        compiler_params=pltpu.CompilerParams(collective_id=0),
    )(dy, W)
